# Initial kernel scaffold; baseline (speedup 1.0000x reference)
import jax
import jax.numpy as jnp
from jax.experimental import pallas as pl


def kernel(inputs, token_table, pos_table):
    raise NotImplementedError("write your pallas kernel here")


# speedup vs baseline: 3.4814x; 3.4814x over previous
import functools

import jax
import jax.numpy as jnp
from jax import lax
from jax.experimental import pallas as pl
from jax.experimental.pallas import tpu as pltpu
from jax.experimental.pallas import tpu_sc as plsc

_NUM_CORES = 2
_NUM_SUBCORES = 16
_NW = _NUM_CORES * _NUM_SUBCORES

_M = 80
_C = 400


def _build(N, D, L):
    per_w = N // _NW
    S = _C // _M
    K = per_w // _C
    IR = per_w // _M

    mesh = plsc.VectorSubcoreMesh(
        core_axis_name="c", subcore_axis_name="s")

    @functools.partial(
        pl.kernel,
        out_type=jax.ShapeDtypeStruct((N, D), jnp.float32),
        mesh=mesh,
        scratch_types=[
            pltpu.VMEM((IR, _M), jnp.int32),
            pltpu.VMEM((_C, D), jnp.float32),
            pltpu.VMEM((_C, D), jnp.float32),
            pltpu.SemaphoreType.DMA,
        ],
        compiler_params=pltpu.CompilerParams(use_tc_tiling_on_sc=False),
    )
    def emb(idx_hbm, tok_hbm, pos_hbm, out_hbm, idx_v, rows_v, posrep_v, sem):
        wid = lax.axis_index("s") * _NUM_CORES + lax.axis_index("c")
        row0 = wid * per_w
        pltpu.sync_copy(idx_hbm.at[pl.ds(wid * IR, IR)], idx_v)
        for r in range(_C // L):
            pltpu.sync_copy(pos_hbm, posrep_v.at[pl.ds(r * L, L)])

        def chunk_body(k, carry):
            cps = [
                pltpu.async_copy(
                    tok_hbm.at[idx_v.at[k * S + j]],
                    rows_v.at[pl.ds(j * _M, _M)],
                    sem,
                )
                for j in range(S)
            ]
            for cp in cps:
                cp.wait()

            def add_body(i, c):
                for t in range(D // 16):
                    plsc.addupdate(
                        rows_v.at[i, pl.ds(t * 16, 16)],
                        posrep_v[i, pl.ds(t * 16, 16)],
                    )
                return c

            lax.fori_loop(0, _C, add_body, 0)
            pltpu.sync_copy(rows_v, out_hbm.at[pl.ds(row0 + k * _C, _C)])
            return carry

        lax.fori_loop(0, K, chunk_body, 0)

    return emb


def kernel(inputs, token_table, pos_table):
    B, L = inputs.shape
    _, D = token_table.shape
    N = B * L
    idx2d = inputs.reshape(N // _M, _M)
    emb = _build(N, D, L)
    out = emb(idx2d, token_table, pos_table)
    return out.reshape(B, L, D)


# speedup vs baseline: 4.2142x; 1.2105x over previous
import functools

import jax
import jax.numpy as jnp
from jax import lax
from jax.experimental import pallas as pl
from jax.experimental.pallas import tpu as pltpu
from jax.experimental.pallas import tpu_sc as plsc

_NUM_CORES = 2
_NUM_SUBCORES = 16
_NW = _NUM_CORES * _NUM_SUBCORES

_M = 100
_NB = 4


def _build(N, D, L):
    per_w = N // _NW
    C = L
    S = C // _M
    K = per_w // C
    IRW = per_w // _M

    mesh = plsc.VectorSubcoreMesh(
        core_axis_name="c", subcore_axis_name="s")

    @functools.partial(
        pl.kernel,
        out_type=jax.ShapeDtypeStruct((N, D), jnp.float32),
        mesh=mesh,
        scratch_types=[
            pltpu.VMEM((IRW, _M), jnp.int32),
            [pltpu.VMEM((C, D), jnp.float32) for _ in range(_NB)],
            pltpu.VMEM((L, D), jnp.float32),
            [pltpu.SemaphoreType.DMA for _ in range(_NB)],
            [pltpu.SemaphoreType.DMA for _ in range(_NB)],
        ],
        compiler_params=pltpu.CompilerParams(use_tc_tiling_on_sc=False),
    )
    def emb(idx_hbm, tok_hbm, pos_hbm, out_hbm,
            idx_v, bufs, pos_v, gsems, osems):
        wid = lax.axis_index("s") * _NUM_CORES + lax.axis_index("c")
        row0 = wid * per_w
        pltpu.sync_copy(idx_hbm.at[pl.ds(wid * IRW, IRW)], idx_v)
        pltpu.sync_copy(pos_hbm, pos_v)

        def fire_gather(c, b):
            for j in range(S):
                pltpu.async_copy(
                    tok_hbm.at[idx_v.at[c * S + j]],
                    bufs[b].at[pl.ds(j * _M, _M)],
                    gsems[b],
                )

        def wait_chunk(b, sem):
            pltpu.make_async_copy(tok_hbm.at[pl.ds(0, C)], bufs[b], sem).wait()

        for c in range(_NB - 1):
            fire_gather(c, c)

        def group(g, carry):
            for i in range(_NB):
                c = g * _NB + i
                wait_chunk(i, gsems[i])

                @plsc.parallel_loop(0, C, unroll=4)
                def _(r):
                    for t in range(D // 16):
                        plsc.addupdate(
                            bufs[i].at[r, pl.ds(t * 16, 16)],
                            pos_v[r, pl.ds(t * 16, 16)],
                        )

                pltpu.async_copy(
                    bufs[i], out_hbm.at[pl.ds(row0 + c * C, C)], osems[i])

                bp = (i + _NB - 1) % _NB

                @pl.when(c >= 1)
                def _():
                    wait_chunk(bp, osems[bp])

                @pl.when(c + _NB - 1 < K)
                def _():
                    fire_gather(c + _NB - 1, bp)

            return carry

        lax.fori_loop(0, K // _NB, group, 0)
        last = (K - 1) % _NB
        wait_chunk(last, osems[last])

    return emb


def kernel(inputs, token_table, pos_table):
    B, L = inputs.shape
    _, D = token_table.shape
    N = B * L
    idx2d = inputs.reshape(N // _M, _M)
    emb = _build(N, D, L)
    out = emb(idx2d, token_table, pos_table)
    return out.reshape(B, L, D)


# speedup vs baseline: 4.2366x; 1.0053x over previous
import functools

import jax
import jax.numpy as jnp
from jax import lax
from jax.experimental import pallas as pl
from jax.experimental.pallas import tpu as pltpu
from jax.experimental.pallas import tpu_sc as plsc

_NUM_CORES = 2
_NUM_SUBCORES = 16
_NW = _NUM_CORES * _NUM_SUBCORES

_M = 40
_NB = 4


def _build(B, N, D, L):
    per_w = N // _NW
    C = L
    S = C // _M
    K = per_w // C

    mesh = plsc.VectorSubcoreMesh(
        core_axis_name="c", subcore_axis_name="s")

    @functools.partial(
        pl.kernel,
        out_type=jax.ShapeDtypeStruct((B, L, D), jnp.float32),
        mesh=mesh,
        scratch_types=[
            pltpu.VMEM((per_w,), jnp.int32),
            [pltpu.VMEM((C, D), jnp.float32) for _ in range(_NB)],
            pltpu.VMEM((L, D), jnp.float32),
            [pltpu.SemaphoreType.DMA for _ in range(_NB)],
            [pltpu.SemaphoreType.DMA for _ in range(_NB)],
        ],
        compiler_params=pltpu.CompilerParams(use_tc_tiling_on_sc=False),
    )
    def emb(idx_hbm, tok_hbm, pos_hbm, out_hbm,
            idx_v, bufs, pos_v, gsems, osems):
        wid = lax.axis_index("s") * _NUM_CORES + lax.axis_index("c")
        b0 = wid * K
        pltpu.sync_copy(idx_hbm.at[pl.ds(wid * per_w, per_w)], idx_v)
        pltpu.sync_copy(pos_hbm, pos_v)

        def fire_gather(c, b):
            for j in range(S):
                pltpu.async_copy(
                    tok_hbm.at[idx_v.at[pl.ds(c * C + j * _M, _M)]],
                    bufs[b].at[pl.ds(j * _M, _M)],
                    gsems[b],
                )

        def wait_chunk(b, sem):
            pltpu.make_async_copy(tok_hbm.at[pl.ds(0, C)], bufs[b], sem).wait()

        for c in range(_NB - 1):
            fire_gather(c, c)

        def group(g, carry):
            for i in range(_NB):
                c = g * _NB + i
                wait_chunk(i, gsems[i])

                @plsc.parallel_loop(0, C, unroll=4)
                def _(r):
                    for t in range(D // 16):
                        plsc.addupdate(
                            bufs[i].at[r, pl.ds(t * 16, 16)],
                            pos_v[r, pl.ds(t * 16, 16)],
                        )

                pltpu.async_copy(
                    bufs[i], out_hbm.at[b0 + c], osems[i])

                bp = (i + _NB - 1) % _NB

                @pl.when(c >= 1)
                def _():
                    wait_chunk(bp, osems[bp])

                @pl.when(c + _NB - 1 < K)
                def _():
                    fire_gather(c + _NB - 1, bp)

            return carry

        lax.fori_loop(0, K // _NB, group, 0)
        last = (K - 1) % _NB
        wait_chunk(last, osems[last])

    return emb


def kernel(inputs, token_table, pos_table):
    B, L = inputs.shape
    _, D = token_table.shape
    N = B * L
    emb = _build(B, N, D, L)
    return emb(inputs.reshape(N), token_table, pos_table)
